# Initial kernel scaffold; baseline (speedup 1.0000x reference)
import jax
import jax.numpy as jnp
from jax.experimental import pallas as pl


def kernel(x, edge_weight, weight, self_weight, bias, gamma, beta, edge_index):
    raise NotImplementedError("write your pallas kernel here")


# speedup vs baseline: 3.8202x; 3.8202x over previous
import functools

import jax
import jax.numpy as jnp
from jax import lax
from jax.experimental import pallas as pl
from jax.experimental.pallas import tpu as pltpu
from jax.experimental.pallas import tpu_sc as plsc

N = 10000
E = 320000
D = 128
NC = 2
NS = 16
NW = NC * NS
B = 128
EPT = ((E // NW + B - 1) // B) * B
EPAD = EPT * NW
ROW_CHUNK = 632


def _sc_spmm_body(src_hbm, dst_hbm, w_hbm, x_hbm, zeros_hbm, out_hbm,
                  src_v, dst_v, w_v, rows_v, agg_sh, sem):
    c = lax.axis_index("c")
    s = lax.axis_index("s")
    wid = s * NC + c

    row_off = s * ROW_CHUNK
    last_off = (NS - 1) * ROW_CHUNK
    last_cnt = N - last_off

    @pl.when(s < NS - 1)
    def _zero_main():
        pltpu.sync_copy(zeros_hbm.at[pl.ds(row_off, ROW_CHUNK)],
                        agg_sh.at[pl.ds(row_off, ROW_CHUNK)])

    @pl.when(s == NS - 1)
    def _zero_last():
        pltpu.sync_copy(zeros_hbm.at[pl.ds(last_off, last_cnt)],
                        agg_sh.at[pl.ds(last_off, last_cnt)])

    plsc.subcore_barrier()

    base = wid * EPT

    def batch_body(b, carry):
        off = base + b * B
        pltpu.sync_copy(src_hbm.at[pl.ds(off, B)], src_v)
        pltpu.sync_copy(dst_hbm.at[pl.ds(off, B)], dst_v)
        pltpu.sync_copy(w_hbm.at[pl.ds(off, B)], w_v)
        pltpu.async_copy(x_hbm.at[src_v], rows_v, sem).wait()

        def group_body(g, carry2):
            w16 = w_v[pl.ds(g * 16, 16)]
            for e16 in range(16):
                e = g * 16 + e16
                wsp = jnp.full((16,), w16[e16], jnp.float32)
                for f in range(D // 16):
                    rows_v[e, pl.ds(f * 16, 16)] = (
                        rows_v[e, pl.ds(f * 16, 16)] * wsp)
            return carry2

        lax.fori_loop(0, B // 16, group_body, 0)

        pltpu.sync_copy(rows_v, agg_sh.at[dst_v], add=True)
        return carry

    lax.fori_loop(0, EPT // B, batch_body, 0)
    plsc.subcore_barrier()

    @pl.when(s < NS - 1)
    def _out_main():
        pltpu.sync_copy(agg_sh.at[pl.ds(row_off, ROW_CHUNK)],
                        out_hbm.at[c, pl.ds(row_off, ROW_CHUNK)])

    @pl.when(s == NS - 1)
    def _out_last():
        pltpu.sync_copy(agg_sh.at[pl.ds(last_off, last_cnt)],
                        out_hbm.at[c, pl.ds(last_off, last_cnt)])


_sc_spmm = functools.partial(
    pl.kernel,
    out_type=jax.ShapeDtypeStruct((NC, N, D), jnp.float32),
    mesh=plsc.VectorSubcoreMesh(core_axis_name="c", subcore_axis_name="s"),
    scratch_types=[
        pltpu.VMEM((B,), jnp.int32),
        pltpu.VMEM((B,), jnp.int32),
        pltpu.VMEM((B,), jnp.float32),
        pltpu.VMEM((B, D), jnp.float32),
        pltpu.VMEM_SHARED((N, D), jnp.float32),
        pltpu.SemaphoreType.DMA,
    ],
)(_sc_spmm_body)


def _tc_body(agg_ref, x_ref, w_ref, sw_ref, bias_ref, gamma_ref, beta_ref,
             out_ref):
    a = agg_ref[0] + agg_ref[1]
    pre = jnp.dot(a, w_ref[...], preferred_element_type=jnp.float32)
    pre = pre + jnp.dot(x_ref[...], sw_ref[...],
                        preferred_element_type=jnp.float32)
    pre = pre + bias_ref[...]
    mean = jnp.mean(pre, axis=0, keepdims=True)
    cen = pre - mean
    var = jnp.mean(cen * cen, axis=0, keepdims=True)
    out_ref[...] = cen * lax.rsqrt(var + 1e-5) * gamma_ref[...] + beta_ref[...]


def kernel(x, edge_weight, weight, self_weight, bias, gamma, beta, edge_index):
    pad = EPAD - E
    dst = jnp.concatenate([edge_index[0], jnp.zeros((pad,), jnp.int32)])
    src = jnp.concatenate([edge_index[1], jnp.zeros((pad,), jnp.int32)])
    w = jnp.concatenate([edge_weight, jnp.zeros((pad,), jnp.float32)])
    zeros = jnp.zeros((N, D), jnp.float32)

    agg = _sc_spmm(src, dst, w, x, zeros)

    out = pl.pallas_call(
        _tc_body,
        out_shape=jax.ShapeDtypeStruct((N, D), jnp.float32),
    )(agg, x, weight, self_weight,
      bias.reshape(1, D), gamma.reshape(1, D), beta.reshape(1, D))
    return out
